# baseline (speedup 1.0000x reference)
import functools

import jax
import jax.numpy as jnp
from jax.experimental import pallas as pl
from jax.experimental.pallas import tpu as pltpu

M_BLOCK = 256
W1_CHUNK = 128


def _gating_kernel(np_, x_ref, w1f_ref, b1_ref, w2f_ref, b2_ref, out_ref,
                   w1b_ref, w2b_ref, acc_ref):
    i = pl.program_id(0)

    def _finish(h_pre):
        h = jnp.maximum(h_pre + b1_ref[...], 0.0).astype(jnp.bfloat16)
        logits = jax.lax.dot_general(
            h, w2b_ref[...], (((1,), (0,)), ((), ())),
            preferred_element_type=jnp.float32)
        logits = logits + b2_ref[...]
        mx = jnp.max(logits, axis=-1, keepdims=True)
        e = jnp.exp(logits - mx)
        out_ref[...] = e / jnp.sum(e, axis=-1, keepdims=True)

    @pl.when(i == 0)
    def _first():
        w2b_ref[...] = w2f_ref[...].astype(jnp.bfloat16)

    @pl.when(i < np_)
    def _warm():
        rows = pl.ds(i * W1_CHUNK, W1_CHUNK)
        w1c = w1f_ref[...].astype(jnp.bfloat16)
        w1b_ref[rows, :] = w1c
        xk = x_ref[:, rows].astype(jnp.bfloat16)
        part = jax.lax.dot_general(
            xk, w1c, (((1,), (0,)), ((), ())),
            preferred_element_type=jnp.float32)

        @pl.when(i == 0)
        def _():
            acc_ref[...] = part

        @pl.when(i > 0)
        def _():
            acc_ref[...] += part

        @pl.when(i == np_ - 1)
        def _():
            _finish(acc_ref[...])

    @pl.when(i >= np_)
    def _main():
        xb = x_ref[...].astype(jnp.bfloat16)
        h_pre = jax.lax.dot_general(
            xb, w1b_ref[...], (((1,), (0,)), ((), ())),
            preferred_element_type=jnp.float32)
        _finish(h_pre)


def kernel(inputs, W1, b1, W2, b2):
    M, K = inputs.shape
    H = W1.shape[1]
    E = W2.shape[1]
    np_ = K // W1_CHUNK
    nm = M // M_BLOCK
    return pl.pallas_call(
        functools.partial(_gating_kernel, np_),
        grid=(np_ + nm - 1,),
        in_specs=[
            pl.BlockSpec((M_BLOCK, K),
                         lambda i: (jnp.maximum(i - np_ + 1, 0), 0)),
            pl.BlockSpec((W1_CHUNK, H),
                         lambda i: (jnp.minimum(i, np_ - 1), 0)),
            pl.BlockSpec((1, H), lambda i: (0, 0)),
            pl.BlockSpec((H, E), lambda i: (0, 0)),
            pl.BlockSpec((1, E), lambda i: (0, 0)),
        ],
        out_specs=pl.BlockSpec((M_BLOCK, E),
                               lambda i: (jnp.maximum(i - np_ + 1, 0), 0)),
        out_shape=jax.ShapeDtypeStruct((M, E), jnp.float32),
        scratch_shapes=[
            pltpu.VMEM((K, H), jnp.bfloat16),
            pltpu.VMEM((H, E), jnp.bfloat16),
            pltpu.VMEM((M_BLOCK, H), jnp.float32),
        ],
        compiler_params=pltpu.CompilerParams(
            dimension_semantics=("arbitrary",),
        ),
    )(inputs, W1, b1.reshape(1, H), W2, b2.reshape(1, E))


# speedup vs baseline: 1.0464x; 1.0464x over previous
import functools

import jax
import jax.numpy as jnp
from jax.experimental import pallas as pl
from jax.experimental.pallas import tpu as pltpu

M_BLOCK = 256
W1_CHUNK = 256


def _gating_kernel(np_, nm, x_ref, w1f_ref, b1_ref, w2f_ref, b2_ref, out_ref,
                   w1b_ref, w2b_ref, xb_ref, acc_ref, lga_ref, lgb_ref):
    i = pl.program_id(0)

    def _store_logits(m, logits):
        @pl.when(jnp.remainder(m, 2) == 0)
        def _():
            lga_ref[...] = logits

        @pl.when(jnp.remainder(m, 2) == 1)
        def _():
            lgb_ref[...] = logits

    @pl.when(i == 0)
    def _first():
        w2b_ref[...] = w2f_ref[...].astype(jnp.bfloat16)
        xb_ref[...] = x_ref[...].astype(jnp.bfloat16)

    @pl.when(i < np_)
    def _warm():
        cols = pl.ds(i * W1_CHUNK, W1_CHUNK)
        w1c = w1f_ref[...].astype(jnp.bfloat16)
        w1b_ref[:, cols] = w1c
        hj = jax.lax.dot_general(
            xb_ref[...], w1c, (((1,), (0,)), ((), ())),
            preferred_element_type=jnp.float32)
        hj = jnp.maximum(hj + b1_ref[:, cols], 0.0).astype(jnp.bfloat16)
        part = jax.lax.dot_general(
            hj, w2b_ref[pl.ds(i * W1_CHUNK, W1_CHUNK), :],
            (((1,), (0,)), ((), ())),
            preferred_element_type=jnp.float32)

        @pl.when(i == 0)
        def _():
            acc_ref[...] = part

        @pl.when(i > 0)
        def _():
            acc_ref[...] += part

        @pl.when(i == np_ - 1)
        def _():
            lga_ref[...] = acc_ref[...]

    @pl.when(i >= np_)
    def _main():
        m_out = i - np_
        m_new = m_out + 1

        @pl.when(m_new <= nm - 1)
        def _compute_new():
            xb = x_ref[...].astype(jnp.bfloat16)
            h = jax.lax.dot_general(
                xb, w1b_ref[...], (((1,), (0,)), ((), ())),
                preferred_element_type=jnp.float32)
            h = jnp.maximum(h + b1_ref[...], 0.0).astype(jnp.bfloat16)
            logits = jax.lax.dot_general(
                h, w2b_ref[...], (((1,), (0,)), ((), ())),
                preferred_element_type=jnp.float32)
            _store_logits(m_new, logits)

        lg = jnp.where(jnp.remainder(m_out, 2) == 0,
                       lga_ref[...], lgb_ref[...])
        lg = lg + b2_ref[...]
        mx = jnp.max(lg, axis=-1, keepdims=True)
        e = jnp.exp(lg - mx)
        out_ref[...] = e / jnp.sum(e, axis=-1, keepdims=True)


def kernel(inputs, W1, b1, W2, b2):
    M, K = inputs.shape
    H = W1.shape[1]
    E = W2.shape[1]
    np_ = H // W1_CHUNK
    nm = M // M_BLOCK
    return pl.pallas_call(
        functools.partial(_gating_kernel, np_, nm),
        grid=(np_ + nm,),
        in_specs=[
            pl.BlockSpec(
                (M_BLOCK, K),
                lambda i: (jnp.clip(i - np_ + 1, 0, nm - 1), 0)),
            pl.BlockSpec((K, W1_CHUNK),
                         lambda i: (0, jnp.minimum(i, np_ - 1))),
            pl.BlockSpec((1, H), lambda i: (0, 0)),
            pl.BlockSpec((H, E), lambda i: (0, 0)),
            pl.BlockSpec((1, E), lambda i: (0, 0)),
        ],
        out_specs=pl.BlockSpec(
            (M_BLOCK, E),
            lambda i: (jnp.clip(i - np_, 0, nm - 1), 0)),
        out_shape=jax.ShapeDtypeStruct((M, E), jnp.float32),
        scratch_shapes=[
            pltpu.VMEM((K, H), jnp.bfloat16),
            pltpu.VMEM((H, E), jnp.bfloat16),
            pltpu.VMEM((M_BLOCK, K), jnp.bfloat16),
            pltpu.VMEM((M_BLOCK, E), jnp.float32),
            pltpu.VMEM((M_BLOCK, E), jnp.float32),
            pltpu.VMEM((M_BLOCK, E), jnp.float32),
        ],
        compiler_params=pltpu.CompilerParams(
            dimension_semantics=("arbitrary",),
        ),
    )(inputs, W1, b1.reshape(1, H), W2, b2.reshape(1, E))


# speedup vs baseline: 1.0499x; 1.0034x over previous
import functools

import jax
import jax.numpy as jnp
from jax.experimental import pallas as pl
from jax.experimental.pallas import tpu as pltpu

M_BLOCK = 256
W1_CHUNK = 256


def _gating_kernel(np_, x_ref, w1f_ref, b1_ref, w2f_ref, b2_ref, out_ref,
                   w1b_ref, w2b_ref, xb_ref, acc_ref):
    i = pl.program_id(0)

    def _softmax_store(logits):
        logits = logits + b2_ref[...]
        mx = jnp.max(logits, axis=-1, keepdims=True)
        e = jnp.exp(logits - mx)
        out_ref[...] = e / jnp.sum(e, axis=-1, keepdims=True)

    @pl.when(i == 0)
    def _first():
        w2b_ref[...] = w2f_ref[...].astype(jnp.bfloat16)
        xb_ref[...] = x_ref[...].astype(jnp.bfloat16)

    @pl.when(i < np_)
    def _warm():
        cols = pl.ds(i * W1_CHUNK, W1_CHUNK)
        w1c = w1f_ref[...].astype(jnp.bfloat16)
        w1b_ref[:, cols] = w1c
        hj = jax.lax.dot_general(
            xb_ref[...], w1c, (((1,), (0,)), ((), ())),
            preferred_element_type=jnp.float32)
        hj = jnp.maximum(hj + b1_ref[:, cols], 0.0).astype(jnp.bfloat16)
        part = jax.lax.dot_general(
            hj, w2b_ref[pl.ds(i * W1_CHUNK, W1_CHUNK), :],
            (((1,), (0,)), ((), ())),
            preferred_element_type=jnp.float32)

        @pl.when(i == 0)
        def _():
            acc_ref[...] = part

        @pl.when(i > 0)
        def _():
            acc_ref[...] += part

        @pl.when(i == np_ - 1)
        def _():
            _softmax_store(acc_ref[...])

    @pl.when(i >= np_)
    def _main():
        xb = x_ref[...].astype(jnp.bfloat16)
        h = jax.lax.dot_general(
            xb, w1b_ref[...], (((1,), (0,)), ((), ())),
            preferred_element_type=jnp.float32)
        h = jnp.maximum(h + b1_ref[...], 0.0).astype(jnp.bfloat16)
        logits = jax.lax.dot_general(
            h, w2b_ref[...], (((1,), (0,)), ((), ())),
            preferred_element_type=jnp.float32)
        _softmax_store(logits)


def kernel(inputs, W1, b1, W2, b2):
    M, K = inputs.shape
    H = W1.shape[1]
    E = W2.shape[1]
    np_ = H // W1_CHUNK
    nm = M // M_BLOCK
    return pl.pallas_call(
        functools.partial(_gating_kernel, np_),
        grid=(np_ + nm - 1,),
        in_specs=[
            pl.BlockSpec((M_BLOCK, K),
                         lambda i: (jnp.maximum(i - np_ + 1, 0), 0)),
            pl.BlockSpec((K, W1_CHUNK),
                         lambda i: (0, jnp.minimum(i, np_ - 1))),
            pl.BlockSpec((1, H), lambda i: (0, 0)),
            pl.BlockSpec((H, E), lambda i: (0, 0)),
            pl.BlockSpec((1, E), lambda i: (0, 0)),
        ],
        out_specs=pl.BlockSpec((M_BLOCK, E),
                               lambda i: (jnp.maximum(i - np_ + 1, 0), 0)),
        out_shape=jax.ShapeDtypeStruct((M, E), jnp.float32),
        scratch_shapes=[
            pltpu.VMEM((K, H), jnp.bfloat16),
            pltpu.VMEM((H, E), jnp.bfloat16),
            pltpu.VMEM((M_BLOCK, K), jnp.bfloat16),
            pltpu.VMEM((M_BLOCK, E), jnp.float32),
        ],
        compiler_params=pltpu.CompilerParams(
            dimension_semantics=("arbitrary",),
        ),
    )(inputs, W1, b1.reshape(1, H), W2, b2.reshape(1, E))
